# baseline (speedup 1.0000x reference)
import functools

import jax
import jax.numpy as jnp
from jax import lax
from jax.experimental import pallas as pl
from jax.experimental.pallas import tpu as pltpu
from jax.experimental.pallas import tpu_sc as plsc

_info = plsc.get_sparse_core_info()
_NC = _info.num_cores
_NS = _info.num_subcores
_NW = _NC * _NS


@functools.partial(jax.jit, static_argnames=("chunk",))
def _sc_gather(table, idx, *, chunk=512):
    B = idx.shape[0]
    D = table.shape[1]
    b_per_w = B // _NW
    n_chunks = b_per_w // chunk
    assert b_per_w * _NW == B and n_chunks * chunk == b_per_w

    mesh = plsc.VectorSubcoreMesh(core_axis_name="c", subcore_axis_name="s")

    @functools.partial(
        pl.kernel,
        out_type=jax.ShapeDtypeStruct((B, D), jnp.float32),
        mesh=mesh,
        scratch_types=[
            pltpu.VMEM((chunk,), jnp.int32),
            pltpu.VMEM((chunk, D), jnp.float32),
            pltpu.SemaphoreType.DMA,
        ],
        compiler_params=pltpu.CompilerParams(use_tc_tiling_on_sc=False),
    )
    def body(table_hbm, idx_hbm, out_hbm, idx_v, rows_v, sem):
        wid = lax.axis_index("s") * _NC + lax.axis_index("c")
        base = wid * b_per_w

        def step(i, carry):
            off = base + i * chunk
            pltpu.sync_copy(idx_hbm.at[pl.ds(off, chunk)], idx_v)
            pltpu.async_copy(table_hbm.at[idx_v], rows_v, sem).wait()
            pltpu.sync_copy(rows_v, out_hbm.at[pl.ds(off, chunk)])
            return carry

        lax.fori_loop(0, n_chunks, step, 0)

    return body(table, idx)


def kernel(x, table):
    assert x.ndim == 4, f"TokenEmbedding expects 4D [B,H,W,C], got {x.shape}"
    vocab, dim = table.shape
    if x.shape[-1] == vocab:
        idx = jnp.argmax(x, axis=-1).astype(jnp.int32)
    else:
        idx = x.astype(jnp.int32)
    B, H, W = x.shape[0], x.shape[1], x.shape[2]
    flat = idx.reshape(-1)
    out = _sc_gather(table, flat)
    return out.reshape(B, H, W, dim)


# speedup vs baseline: 1.0286x; 1.0286x over previous
import functools

import jax
import jax.numpy as jnp
from jax import lax
from jax.experimental import pallas as pl
from jax.experimental.pallas import tpu as pltpu
from jax.experimental.pallas import tpu_sc as plsc

_info = plsc.get_sparse_core_info()
_NC = _info.num_cores
_NS = _info.num_subcores
_NW = _NC * _NS


@functools.partial(jax.jit, static_argnames=("chunk", "nbuf"))
def _sc_gather(table, idx, *, chunk=640, nbuf=2):
    B = idx.shape[0]
    D = table.shape[1]
    b_per_w = B // _NW
    n_chunks = b_per_w // chunk
    n_groups = n_chunks // nbuf
    assert b_per_w * _NW == B and n_groups * nbuf * chunk == b_per_w

    mesh = plsc.VectorSubcoreMesh(core_axis_name="c", subcore_axis_name="s")

    @functools.partial(
        pl.kernel,
        out_type=jax.ShapeDtypeStruct((B, D), jnp.float32),
        mesh=mesh,
        scratch_types=(
            [pltpu.VMEM((b_per_w,), jnp.int32)]
            + [pltpu.VMEM((chunk, D), jnp.float32) for _ in range(nbuf)]
            + [pltpu.SemaphoreType.DMA for _ in range(2 * nbuf)]
        ),
        compiler_params=pltpu.CompilerParams(use_tc_tiling_on_sc=False),
    )
    def body(table_hbm, idx_hbm, out_hbm, idx_all, *bufs):
        rows = bufs[:nbuf]
        gsem = bufs[nbuf:2 * nbuf]
        wsem = bufs[2 * nbuf:]
        wid = lax.axis_index("s") * _NC + lax.axis_index("c")
        base = wid * b_per_w

        pltpu.sync_copy(idx_hbm.at[pl.ds(base, b_per_w)], idx_all)

        def start_gather(i, b):
            src = table_hbm.at[idx_all.at[pl.ds(i * chunk, chunk)]]
            return pltpu.async_copy(src, rows[b], gsem[b])

        for b in range(nbuf):
            start_gather(b, b)

        def group(g, carry):
            for b in range(nbuf):
                i = g * nbuf + b
                pltpu.make_async_copy(
                    table_hbm.at[idx_all.at[pl.ds(i * chunk, chunk)]],
                    rows[b], gsem[b]).wait()
                pltpu.async_copy(
                    rows[b], out_hbm.at[pl.ds(base + i * chunk, chunk)],
                    wsem[b])

                @pl.when(g < n_groups - 1)
                def _():
                    pltpu.make_async_copy(
                        rows[b], out_hbm.at[pl.ds(base + i * chunk, chunk)],
                        wsem[b]).wait()
                    start_gather(i + nbuf, b)
            return carry

        lax.fori_loop(0, n_groups, group, 0)

        for b in range(nbuf):
            i = (n_groups - 1) * nbuf + b
            pltpu.make_async_copy(
                rows[b], out_hbm.at[pl.ds(base + i * chunk, chunk)],
                wsem[b]).wait()

    return body(table, idx)


def kernel(x, table):
    assert x.ndim == 4, f"TokenEmbedding expects 4D [B,H,W,C], got {x.shape}"
    vocab, dim = table.shape
    if x.shape[-1] == vocab:
        idx = jnp.argmax(x, axis=-1).astype(jnp.int32)
    else:
        idx = x.astype(jnp.int32)
    B, H, W = x.shape[0], x.shape[1], x.shape[2]
    flat = idx.reshape(-1)
    out = _sc_gather(table, flat)
    return out.reshape(B, H, W, dim)


# speedup vs baseline: 1.0531x; 1.0237x over previous
import functools

import jax
import jax.numpy as jnp
from jax import lax
from jax.experimental import pallas as pl
from jax.experimental.pallas import tpu as pltpu
from jax.experimental.pallas import tpu_sc as plsc

_info = plsc.get_sparse_core_info()
_NC = _info.num_cores
_NS = _info.num_subcores
_NW = _NC * _NS


@functools.partial(jax.jit, static_argnames=("cell", "nbuf"))
def _sc_gather(table, idx, *, cell=400, nbuf=2):
    B = idx.shape[0]
    D = table.shape[1]
    NB = B // cell
    b_per_w = NB // _NW
    n_groups = b_per_w // nbuf
    assert NB * cell == B and n_groups * nbuf == b_per_w

    mesh = plsc.VectorSubcoreMesh(core_axis_name="c", subcore_axis_name="s")

    @functools.partial(
        pl.kernel,
        out_type=jax.ShapeDtypeStruct((NB, cell, D), jnp.float32),
        mesh=mesh,
        scratch_types=(
            [pltpu.VMEM((b_per_w * cell,), jnp.int32)]
            + [pltpu.VMEM((cell, D), jnp.float32) for _ in range(nbuf)]
            + [pltpu.SemaphoreType.DMA for _ in range(2 * nbuf)]
        ),
        compiler_params=pltpu.CompilerParams(use_tc_tiling_on_sc=False),
    )
    def body(table_hbm, idx_hbm, out_hbm, idx_all, *bufs):
        rows = bufs[:nbuf]
        gsem = bufs[nbuf:2 * nbuf]
        wsem = bufs[2 * nbuf:]
        wid = lax.axis_index("s") * _NC + lax.axis_index("c")
        base = wid * b_per_w

        pltpu.sync_copy(idx_hbm.at[pl.ds(base * cell, b_per_w * cell)],
                        idx_all)

        def start_gather(i, b):
            src = table_hbm.at[idx_all.at[pl.ds(i * cell, cell)]]
            return pltpu.async_copy(src, rows[b], gsem[b])

        for b in range(nbuf):
            start_gather(b, b)

        def group(g, carry):
            for b in range(nbuf):
                i = g * nbuf + b
                pltpu.make_async_copy(
                    table_hbm.at[idx_all.at[pl.ds(i * cell, cell)]],
                    rows[b], gsem[b]).wait()
                pltpu.async_copy(rows[b], out_hbm.at[base + i], wsem[b])

                @pl.when(g < n_groups - 1)
                def _():
                    pltpu.make_async_copy(
                        rows[b], out_hbm.at[base + i], wsem[b]).wait()
                    start_gather(i + nbuf, b)
            return carry

        lax.fori_loop(0, n_groups, group, 0)

        for b in range(nbuf):
            i = (n_groups - 1) * nbuf + b
            pltpu.make_async_copy(
                rows[b], out_hbm.at[base + i], wsem[b]).wait()

    return body(table, idx)


def kernel(x, table):
    assert x.ndim == 4, f"TokenEmbedding expects 4D [B,H,W,C], got {x.shape}"
    vocab, dim = table.shape
    if x.shape[-1] == vocab:
        idx = jnp.argmax(x, axis=-1).astype(jnp.int32)
    else:
        idx = x.astype(jnp.int32)
    B, H, W = x.shape[0], x.shape[1], x.shape[2]
    flat = idx.reshape(-1)
    out3 = _sc_gather(table, flat, cell=H * W)
    return out3.reshape(B, H, W, dim)
